# baseline (speedup 1.0000x reference)
import functools

import jax
import jax.numpy as jnp
from jax import lax
from jax.experimental import pallas as pl
from jax.experimental.pallas import tpu as pltpu
from jax.experimental.pallas import tpu_sc as plsc

N = 10000
C = 128
E = 320000
NC = 2
NS = 16
NW = NC * NS
NP = 10240
RPT = NP // NS
CH = 128
CPW = 80
EPW = CPW * CH
E_PAD = NW * EPW
PAD_DST = N + 16
RB = 10
RBS = N // RB

_mesh = plsc.VectorSubcoreMesh(
    core_axis_name="c", subcore_axis_name="s", num_cores=NC, num_subcores=NS
)


def _wid():
    return lax.axis_index("s") * NC + lax.axis_index("c")


@functools.partial(
    pl.kernel,
    out_type=jax.ShapeDtypeStruct((NC, NP), jnp.float32),
    mesh=_mesh,
    scratch_types=[
        pltpu.VMEM((CPW, CH), jnp.int32),
        pltpu.VMEM((CH,), jnp.float32),
        pltpu.VMEM((RPT,), jnp.float32),
        pltpu.VMEM_SHARED((NP,), jnp.float32),
    ],
)
def _deg_kernel(dst2d, degp, idx_v, ones_v, zero_v, deg_sh):
    cid = lax.axis_index("c")
    sid = lax.axis_index("s")
    wid = _wid()
    for k in range(CH // 16):
        ones_v[pl.ds(k * 16, 16)] = jnp.ones((16,), jnp.float32)
    for k in range(RPT // 16):
        zero_v[pl.ds(k * 16, 16)] = jnp.zeros((16,), jnp.float32)
    pltpu.sync_copy(zero_v, deg_sh.at[pl.ds(sid * RPT, RPT)])
    pltpu.sync_copy(dst2d.at[pl.ds(wid * CPW, CPW)], idx_v)
    plsc.subcore_barrier()

    @pl.loop(0, CPW)
    def _chunk(j):
        pltpu.sync_copy(ones_v, deg_sh.at[idx_v.at[j]], add=True)

    plsc.subcore_barrier()
    pltpu.sync_copy(
        deg_sh.at[pl.ds(sid * RPT, RPT)], degp.at[cid, pl.ds(sid * RPT, RPT)]
    )


def _matmul_body(x_ref, w_ref, xw_ref):
    xw_ref[...] = jnp.dot(
        x_ref[...], w_ref[...], preferred_element_type=jnp.float32
    )


_matmul = pl.pallas_call(
    _matmul_body,
    grid=(RB,),
    in_specs=[
        pl.BlockSpec((RBS, C), lambda i: (i, 0)),
        pl.BlockSpec((C, C), lambda i: (0, 0)),
    ],
    out_specs=pl.BlockSpec((RBS, C), lambda i: (i, 0)),
    out_shape=jax.ShapeDtypeStruct((N, C), jnp.float32),
)


def _scale_body(xw_ref, degp_ref, y_ref):
    deg = degp_ref[0, :, 0] + degp_ref[1, :, 0] + 1.0
    dinv = lax.rsqrt(deg)
    y_ref[...] = xw_ref[...] * dinv[:, None]


_scale = pl.pallas_call(
    _scale_body,
    grid=(RB,),
    in_specs=[
        pl.BlockSpec((RBS, C), lambda i: (i, 0)),
        pl.BlockSpec((NC, RBS, 1), lambda i: (0, i, 0)),
    ],
    out_specs=pl.BlockSpec((RBS, C), lambda i: (i, 0)),
    out_shape=jax.ShapeDtypeStruct((N, C), jnp.float32),
)


Q = 8
NG = CPW // Q
NGP = NG // 2


@functools.partial(
    pl.kernel,
    out_type=jax.ShapeDtypeStruct((NC, NP, C), jnp.float32),
    mesh=_mesh,
    scratch_types=[
        pltpu.VMEM((CPW, CH), jnp.int32),
        pltpu.VMEM((2, Q, CH), jnp.int32),
        pltpu.VMEM((2, CH, C), jnp.float32),
        pltpu.VMEM_SHARED((NP, C), jnp.float32),
        pltpu.SemaphoreType.DMA,
        pltpu.SemaphoreType.DMA,
        pltpu.SemaphoreType.DMA,
    ],
)
def _agg_kernel(y_hbm, src2d, dst2d, accp, sib, dib, rows, acc_sh,
                gsem0, gsem1, isem):
    cid = lax.axis_index("c")
    sid = lax.axis_index("s")
    wid = _wid()
    base = wid * CPW
    gsem = (gsem0, gsem1)

    @pl.loop(0, CH)
    def _zrow(i):
        for k in range(C // 16):
            rows[0, i, pl.ds(k * 16, 16)] = jnp.zeros((16,), jnp.float32)

    for k in range(RPT // CH):
        pltpu.sync_copy(
            rows.at[0], acc_sh.at[pl.ds(sid * RPT + k * CH, CH), :]
        )
    pltpu.sync_copy(src2d.at[pl.ds(base, CPW)], sib)
    pltpu.sync_copy(dst2d.at[pl.ds(base, Q)], dib.at[0])
    plsc.subcore_barrier()
    pltpu.async_copy(y_hbm.at[sib.at[0]], rows.at[0], gsem0)

    @pl.loop(0, NGP)
    def _pair(gp):
        for h in range(2):
            g = gp * 2 + h
            if h == 1:
                pltpu.make_async_copy(
                    dst2d.at[pl.ds(base + g * Q, Q)], dib.at[1], isem
                ).wait()
            else:
                @pl.when(g > 0)
                def _():
                    pltpu.make_async_copy(
                        dst2d.at[pl.ds(base + g * Q, Q)], dib.at[0], isem
                    ).wait()
            @pl.when(g + 1 < NG)
            def _():
                pltpu.async_copy(
                    dst2d.at[pl.ds(base + (g + 1) * Q, Q)],
                    dib.at[(h + 1) % 2], isem,
                )
            for c in range(Q):
                t = g * Q + c
                if c < Q - 1:
                    pltpu.async_copy(
                        y_hbm.at[sib.at[t + 1]],
                        rows.at[(c + 1) % 2], gsem[(c + 1) % 2],
                    )
                else:
                    @pl.when(g + 1 < NG)
                    def _():
                        pltpu.async_copy(
                            y_hbm.at[sib.at[t + 1]],
                            rows.at[(c + 1) % 2], gsem[(c + 1) % 2],
                        )
                pltpu.make_async_copy(
                    y_hbm.at[sib.at[t]], rows.at[c % 2], gsem[c % 2]
                ).wait()
                pltpu.sync_copy(
                    rows.at[c % 2], acc_sh.at[dib.at[h, c]], add=True
                )

    plsc.subcore_barrier()
    for k in range(RPT // CH):
        pltpu.sync_copy(
            acc_sh.at[pl.ds(sid * RPT + k * CH, CH), :],
            accp.at[cid, pl.ds(sid * RPT + k * CH, CH), :],
        )


def _combine_body(accp_ref, y_ref, degp_ref, b_ref, o_ref):
    deg = degp_ref[0, :, 0] + degp_ref[1, :, 0] + 1.0
    dinv = lax.rsqrt(deg)
    s = accp_ref[0] + accp_ref[1] + y_ref[...]
    o_ref[...] = s * dinv[:, None] + b_ref[...]


_combine = pl.pallas_call(
    _combine_body,
    grid=(RB,),
    in_specs=[
        pl.BlockSpec((NC, RBS, C), lambda i: (0, i, 0)),
        pl.BlockSpec((RBS, C), lambda i: (i, 0)),
        pl.BlockSpec((NC, RBS, 1), lambda i: (0, i, 0)),
        pl.BlockSpec((1, C), lambda i: (0, 0)),
    ],
    out_specs=pl.BlockSpec((RBS, C), lambda i: (i, 0)),
    out_shape=jax.ShapeDtypeStruct((N, C), jnp.float32),
)


def kernel(x, edge_index, W, b):
    src = edge_index[0].astype(jnp.int32)
    dst = edge_index[1].astype(jnp.int32)
    pad = jnp.arange(E_PAD - E, dtype=jnp.int32)
    src2d = jnp.concatenate([src, pad % N]).reshape(E_PAD // CH, CH)
    dst2d = jnp.concatenate(
        [dst, N + pad % (NP - N)]
    ).reshape(E_PAD // CH, CH)
    degp = _deg_kernel(dst2d)
    xw = _matmul(x, W)
    degp3 = degp.reshape(NC, NP, 1)
    y = _scale(xw, degp3)
    accp = _agg_kernel(y, src2d, dst2d)
    return _combine(accp, y, degp3, b.reshape(1, C))


# speedup vs baseline: 1.0016x; 1.0016x over previous
import functools

import jax
import jax.numpy as jnp
from jax import lax
from jax.experimental import pallas as pl
from jax.experimental.pallas import tpu as pltpu
from jax.experimental.pallas import tpu_sc as plsc

N = 10000
C = 128
E = 320000
NC = 2
NS = 16
NW = NC * NS
NP = 10240
RPT = NP // NS
CH = 128
CPW = 80
EPW = CPW * CH
E_PAD = NW * EPW
PAD_DST = N + 16
RB = 10
RBS = N // RB

_mesh = plsc.VectorSubcoreMesh(
    core_axis_name="c", subcore_axis_name="s", num_cores=NC, num_subcores=NS
)


def _wid():
    return lax.axis_index("s") * NC + lax.axis_index("c")


@functools.partial(
    pl.kernel,
    out_type=jax.ShapeDtypeStruct((NC, NP), jnp.float32),
    mesh=_mesh,
    scratch_types=[
        pltpu.VMEM((CPW, CH), jnp.int32),
        pltpu.VMEM((CH,), jnp.float32),
        pltpu.VMEM((RPT,), jnp.float32),
        pltpu.VMEM_SHARED((NP,), jnp.float32),
    ],
)
def _deg_kernel(dst2d, degp, idx_v, ones_v, zero_v, deg_sh):
    cid = lax.axis_index("c")
    sid = lax.axis_index("s")
    wid = _wid()
    for k in range(CH // 16):
        ones_v[pl.ds(k * 16, 16)] = jnp.ones((16,), jnp.float32)
    for k in range(RPT // 16):
        zero_v[pl.ds(k * 16, 16)] = jnp.zeros((16,), jnp.float32)
    pltpu.sync_copy(zero_v, deg_sh.at[pl.ds(sid * RPT, RPT)])
    pltpu.sync_copy(dst2d.at[pl.ds(wid * CPW, CPW)], idx_v)
    plsc.subcore_barrier()

    @pl.loop(0, CPW)
    def _chunk(j):
        pltpu.sync_copy(ones_v, deg_sh.at[idx_v.at[j]], add=True)

    plsc.subcore_barrier()
    pltpu.sync_copy(
        deg_sh.at[pl.ds(sid * RPT, RPT)], degp.at[cid, pl.ds(sid * RPT, RPT)]
    )


def _matmul_body(x_ref, w_ref, xw_ref):
    xw_ref[...] = jnp.dot(
        x_ref[...], w_ref[...], preferred_element_type=jnp.float32
    )


_matmul = pl.pallas_call(
    _matmul_body,
    grid=(RB,),
    in_specs=[
        pl.BlockSpec((RBS, C), lambda i: (i, 0)),
        pl.BlockSpec((C, C), lambda i: (0, 0)),
    ],
    out_specs=pl.BlockSpec((RBS, C), lambda i: (i, 0)),
    out_shape=jax.ShapeDtypeStruct((N, C), jnp.float32),
)


def _scale_body(xw_ref, degp_ref, y_ref):
    deg = degp_ref[0, :, 0] + degp_ref[1, :, 0] + 1.0
    dinv = lax.rsqrt(deg)
    y_ref[...] = xw_ref[...] * dinv[:, None]


_scale = pl.pallas_call(
    _scale_body,
    grid=(RB,),
    in_specs=[
        pl.BlockSpec((RBS, C), lambda i: (i, 0)),
        pl.BlockSpec((NC, RBS, 1), lambda i: (0, i, 0)),
    ],
    out_specs=pl.BlockSpec((RBS, C), lambda i: (i, 0)),
    out_shape=jax.ShapeDtypeStruct((N, C), jnp.float32),
)


Q = 8
NG = CPW // Q
NGP = NG // 2


@functools.partial(
    pl.kernel,
    out_type=jax.ShapeDtypeStruct((NC, NP, C), jnp.float32),
    mesh=_mesh,
    scratch_types=[
        pltpu.VMEM((CPW, CH), jnp.int32),
        pltpu.VMEM((2, Q, CH), jnp.int32),
        pltpu.VMEM((2, CH, C), jnp.float32),
        pltpu.VMEM_SHARED((NP, C), jnp.float32),
        pltpu.SemaphoreType.DMA,
        pltpu.SemaphoreType.DMA,
        pltpu.SemaphoreType.DMA,
        pltpu.SemaphoreType.DMA,
        pltpu.SemaphoreType.DMA,
    ],
)
def _agg_kernel(y_hbm, src2d, dst2d, accp, sib, dib, rows, acc_sh,
                gsem0, gsem1, isem, ssem0, ssem1):
    cid = lax.axis_index("c")
    sid = lax.axis_index("s")
    wid = _wid()
    base = wid * CPW
    gsem = (gsem0, gsem1)
    ssem = (ssem0, ssem1)

    @pl.loop(0, CH)
    def _zrow(i):
        for k in range(C // 16):
            rows[0, i, pl.ds(k * 16, 16)] = jnp.zeros((16,), jnp.float32)

    for k in range(RPT // CH):
        pltpu.sync_copy(
            rows.at[0], acc_sh.at[pl.ds(sid * RPT + k * CH, CH), :]
        )
    pltpu.sync_copy(src2d.at[pl.ds(base, CPW)], sib)
    pltpu.sync_copy(dst2d.at[pl.ds(base, Q)], dib.at[0])
    plsc.subcore_barrier()
    pltpu.async_copy(y_hbm.at[sib.at[0]], rows.at[0], gsem0)

    @pl.loop(0, NGP)
    def _pair(gp):
        for h in range(2):
            g = gp * 2 + h
            @pl.when(g > 0)
            def _():
                pltpu.make_async_copy(
                    rows.at[1],
                    acc_sh.at[dib.at[(h + 1) % 2, Q - 1]], ssem[1]
                ).wait()
            if h == 1:
                pltpu.make_async_copy(
                    dst2d.at[pl.ds(base + g * Q, Q)], dib.at[1], isem
                ).wait()
            else:
                @pl.when(g > 0)
                def _():
                    pltpu.make_async_copy(
                        dst2d.at[pl.ds(base + g * Q, Q)], dib.at[0], isem
                    ).wait()
            @pl.when(g + 1 < NG)
            def _():
                pltpu.async_copy(
                    dst2d.at[pl.ds(base + (g + 1) * Q, Q)],
                    dib.at[(h + 1) % 2], isem,
                )
            for c in range(Q):
                t = g * Q + c
                if c > 0:
                    pltpu.make_async_copy(
                        rows.at[(c - 1) % 2],
                        acc_sh.at[dib.at[h, c - 1]], ssem[(c - 1) % 2]
                    ).wait()
                if c < Q - 1:
                    pltpu.async_copy(
                        y_hbm.at[sib.at[t + 1]],
                        rows.at[(c + 1) % 2], gsem[(c + 1) % 2],
                    )
                else:
                    @pl.when(g + 1 < NG)
                    def _():
                        pltpu.async_copy(
                            y_hbm.at[sib.at[t + 1]],
                            rows.at[(c + 1) % 2], gsem[(c + 1) % 2],
                        )
                pltpu.make_async_copy(
                    y_hbm.at[sib.at[t]], rows.at[c % 2], gsem[c % 2]
                ).wait()
                pltpu.async_copy(
                    rows.at[c % 2], acc_sh.at[dib.at[h, c]],
                    ssem[c % 2], add=True,
                )

    pltpu.make_async_copy(
        rows.at[1], acc_sh.at[dib.at[1, Q - 1]], ssem[1]
    ).wait()
    plsc.subcore_barrier()
    for k in range(RPT // CH):
        pltpu.sync_copy(
            acc_sh.at[pl.ds(sid * RPT + k * CH, CH), :],
            accp.at[cid, pl.ds(sid * RPT + k * CH, CH), :],
        )


def _combine_body(accp_ref, y_ref, degp_ref, b_ref, o_ref):
    deg = degp_ref[0, :, 0] + degp_ref[1, :, 0] + 1.0
    dinv = lax.rsqrt(deg)
    s = accp_ref[0] + accp_ref[1] + y_ref[...]
    o_ref[...] = s * dinv[:, None] + b_ref[...]


_combine = pl.pallas_call(
    _combine_body,
    grid=(RB,),
    in_specs=[
        pl.BlockSpec((NC, RBS, C), lambda i: (0, i, 0)),
        pl.BlockSpec((RBS, C), lambda i: (i, 0)),
        pl.BlockSpec((NC, RBS, 1), lambda i: (0, i, 0)),
        pl.BlockSpec((1, C), lambda i: (0, 0)),
    ],
    out_specs=pl.BlockSpec((RBS, C), lambda i: (i, 0)),
    out_shape=jax.ShapeDtypeStruct((N, C), jnp.float32),
)


def kernel(x, edge_index, W, b):
    src = edge_index[0].astype(jnp.int32)
    dst = edge_index[1].astype(jnp.int32)
    pad = jnp.arange(E_PAD - E, dtype=jnp.int32)
    src2d = jnp.concatenate([src, pad % N]).reshape(E_PAD // CH, CH)
    dst2d = jnp.concatenate(
        [dst, N + pad % (NP - N)]
    ).reshape(E_PAD // CH, CH)
    degp = _deg_kernel(dst2d)
    xw = _matmul(x, W)
    degp3 = degp.reshape(NC, NP, 1)
    y = _scale(xw, degp3)
    accp = _agg_kernel(y, src2d, dst2d)
    return _combine(accp, y, degp3, b.reshape(1, C))


# speedup vs baseline: 1.0390x; 1.0374x over previous
import functools

import jax
import jax.numpy as jnp
from jax import lax
from jax.experimental import pallas as pl
from jax.experimental.pallas import tpu as pltpu
from jax.experimental.pallas import tpu_sc as plsc

N = 10000
C = 128
E = 320000
NC = 2
NS = 16
NW = NC * NS
NP = 10240
RPT = NP // NS
CH = 128
CPW = 80
EPW = CPW * CH
E_PAD = NW * EPW
PAD_DST = N + 16
RB = 10
RBS = N // RB

_mesh = plsc.VectorSubcoreMesh(
    core_axis_name="c", subcore_axis_name="s", num_cores=NC, num_subcores=NS
)


def _wid():
    return lax.axis_index("s") * NC + lax.axis_index("c")


@functools.partial(
    pl.kernel,
    out_type=jax.ShapeDtypeStruct((NC, NP), jnp.float32),
    mesh=_mesh,
    scratch_types=[
        pltpu.VMEM((CPW, CH), jnp.int32),
        pltpu.VMEM((CH,), jnp.float32),
        pltpu.VMEM((RPT,), jnp.float32),
        pltpu.VMEM_SHARED((NP,), jnp.float32),
        pltpu.SemaphoreType.DMA,
    ],
)
def _deg_kernel(dst2d, degp, idx_v, ones_v, zero_v, deg_sh, dsem):
    cid = lax.axis_index("c")
    sid = lax.axis_index("s")
    wid = _wid()
    for k in range(CH // 16):
        ones_v[pl.ds(k * 16, 16)] = jnp.ones((16,), jnp.float32)
    for k in range(RPT // 16):
        zero_v[pl.ds(k * 16, 16)] = jnp.zeros((16,), jnp.float32)
    pltpu.sync_copy(zero_v, deg_sh.at[pl.ds(sid * RPT, RPT)])
    pltpu.sync_copy(dst2d.at[pl.ds(wid * CPW, CPW)], idx_v)
    plsc.subcore_barrier()

    @pl.loop(0, CPW)
    def _chunk(j):
        pltpu.async_copy(ones_v, deg_sh.at[idx_v.at[j]], dsem, add=True)

    @pl.loop(0, CPW)
    def _drain(j):
        pltpu.make_async_copy(
            ones_v, deg_sh.at[idx_v.at[j]], dsem
        ).wait()

    plsc.subcore_barrier()
    pltpu.sync_copy(
        deg_sh.at[pl.ds(sid * RPT, RPT)], degp.at[cid, pl.ds(sid * RPT, RPT)]
    )


def _linear_body(x_ref, w_ref, degp_ref, y_ref):
    deg = degp_ref[0, :, 0] + degp_ref[1, :, 0] + 1.0
    dinv = lax.rsqrt(deg)
    xw = jnp.dot(x_ref[...], w_ref[...], preferred_element_type=jnp.float32)
    y_ref[...] = xw * dinv[:, None]


_linear = pl.pallas_call(
    _linear_body,
    grid=(RB,),
    in_specs=[
        pl.BlockSpec((RBS, C), lambda i: (i, 0)),
        pl.BlockSpec((C, C), lambda i: (0, 0)),
        pl.BlockSpec((NC, RBS, 1), lambda i: (0, i, 0)),
    ],
    out_specs=pl.BlockSpec((RBS, C), lambda i: (i, 0)),
    out_shape=jax.ShapeDtypeStruct((N, C), jnp.float32),
)


Q = 8
NG = CPW // Q
NGP = NG // 2


@functools.partial(
    pl.kernel,
    out_type=jax.ShapeDtypeStruct((NC, NP, C), jnp.float32),
    mesh=_mesh,
    scratch_types=[
        pltpu.VMEM((CPW, CH), jnp.int32),
        pltpu.VMEM((2, Q, CH), jnp.int32),
        pltpu.VMEM((2, CH, C), jnp.float32),
        pltpu.VMEM_SHARED((NP, C), jnp.float32),
        pltpu.SemaphoreType.DMA,
        pltpu.SemaphoreType.DMA,
        pltpu.SemaphoreType.DMA,
        pltpu.SemaphoreType.DMA,
        pltpu.SemaphoreType.DMA,
    ],
)
def _agg_kernel(y_hbm, src2d, dst2d, accp, sib, dib, rows, acc_sh,
                gsem0, gsem1, isem, ssem0, ssem1):
    cid = lax.axis_index("c")
    sid = lax.axis_index("s")
    wid = _wid()
    base = wid * CPW
    gsem = (gsem0, gsem1)
    ssem = (ssem0, ssem1)

    @pl.loop(0, CH)
    def _zrow(i):
        for k in range(C // 16):
            rows[0, i, pl.ds(k * 16, 16)] = jnp.zeros((16,), jnp.float32)

    for k in range(RPT // CH):
        pltpu.async_copy(
            rows.at[0], acc_sh.at[pl.ds(sid * RPT + k * CH, CH), :], ssem0
        )
    pltpu.sync_copy(src2d.at[pl.ds(base, CPW)], sib)
    pltpu.sync_copy(dst2d.at[pl.ds(base, Q)], dib.at[0])
    for k in range(RPT // CH):
        pltpu.make_async_copy(
            rows.at[0], acc_sh.at[pl.ds(sid * RPT + k * CH, CH), :], ssem0
        ).wait()
    plsc.subcore_barrier()
    pltpu.async_copy(y_hbm.at[sib.at[0]], rows.at[0], gsem0)

    @pl.loop(0, NGP)
    def _pair(gp):
        for h in range(2):
            g = gp * 2 + h
            @pl.when(g > 0)
            def _():
                pltpu.make_async_copy(
                    rows.at[1],
                    acc_sh.at[dib.at[(h + 1) % 2, Q - 1]], ssem[1]
                ).wait()
            if h == 1:
                pltpu.make_async_copy(
                    dst2d.at[pl.ds(base + g * Q, Q)], dib.at[1], isem
                ).wait()
            else:
                @pl.when(g > 0)
                def _():
                    pltpu.make_async_copy(
                        dst2d.at[pl.ds(base + g * Q, Q)], dib.at[0], isem
                    ).wait()
            @pl.when(g + 1 < NG)
            def _():
                pltpu.async_copy(
                    dst2d.at[pl.ds(base + (g + 1) * Q, Q)],
                    dib.at[(h + 1) % 2], isem,
                )
            for c in range(Q):
                t = g * Q + c
                if c > 0:
                    pltpu.make_async_copy(
                        rows.at[(c - 1) % 2],
                        acc_sh.at[dib.at[h, c - 1]], ssem[(c - 1) % 2]
                    ).wait()
                if c < Q - 1:
                    pltpu.async_copy(
                        y_hbm.at[sib.at[t + 1]],
                        rows.at[(c + 1) % 2], gsem[(c + 1) % 2],
                    )
                else:
                    @pl.when(g + 1 < NG)
                    def _():
                        pltpu.async_copy(
                            y_hbm.at[sib.at[t + 1]],
                            rows.at[(c + 1) % 2], gsem[(c + 1) % 2],
                        )
                pltpu.make_async_copy(
                    y_hbm.at[sib.at[t]], rows.at[c % 2], gsem[c % 2]
                ).wait()
                pltpu.async_copy(
                    rows.at[c % 2], acc_sh.at[dib.at[h, c]],
                    ssem[c % 2], add=True,
                )

    pltpu.make_async_copy(
        rows.at[1], acc_sh.at[dib.at[1, Q - 1]], ssem[1]
    ).wait()
    plsc.subcore_barrier()
    for k in range(RPT // CH):
        pltpu.async_copy(
            acc_sh.at[pl.ds(sid * RPT + k * CH, CH), :],
            accp.at[cid, pl.ds(sid * RPT + k * CH, CH), :],
            gsem0,
        )
    for k in range(RPT // CH):
        pltpu.make_async_copy(
            acc_sh.at[pl.ds(sid * RPT + k * CH, CH), :],
            accp.at[cid, pl.ds(sid * RPT + k * CH, CH), :],
            gsem0,
        ).wait()


def _combine_body(accp_ref, y_ref, degp_ref, b_ref, o_ref):
    deg = degp_ref[0, :, 0] + degp_ref[1, :, 0] + 1.0
    dinv = lax.rsqrt(deg)
    s = accp_ref[0] + accp_ref[1] + y_ref[...]
    o_ref[...] = s * dinv[:, None] + b_ref[...]


_combine = pl.pallas_call(
    _combine_body,
    grid=(RB,),
    in_specs=[
        pl.BlockSpec((NC, RBS, C), lambda i: (0, i, 0)),
        pl.BlockSpec((RBS, C), lambda i: (i, 0)),
        pl.BlockSpec((NC, RBS, 1), lambda i: (0, i, 0)),
        pl.BlockSpec((1, C), lambda i: (0, 0)),
    ],
    out_specs=pl.BlockSpec((RBS, C), lambda i: (i, 0)),
    out_shape=jax.ShapeDtypeStruct((N, C), jnp.float32),
)


def kernel(x, edge_index, W, b):
    src = edge_index[0].astype(jnp.int32)
    dst = edge_index[1].astype(jnp.int32)
    pad = jnp.arange(E_PAD - E, dtype=jnp.int32)
    src2d = jnp.concatenate([src, pad % N]).reshape(E_PAD // CH, CH)
    dst2d = jnp.concatenate(
        [dst, N + pad % (NP - N)]
    ).reshape(E_PAD // CH, CH)
    degp = _deg_kernel(dst2d)
    degp3 = degp.reshape(NC, NP, 1)
    y = _linear(x, W, degp3)
    accp = _agg_kernel(y, src2d, dst2d)
    return _combine(accp, y, degp3, b.reshape(1, C))


# speedup vs baseline: 1.0678x; 1.0277x over previous
import functools

import jax
import jax.numpy as jnp
from jax import lax
from jax.experimental import pallas as pl
from jax.experimental.pallas import tpu as pltpu
from jax.experimental.pallas import tpu_sc as plsc

N = 10000
C = 128
E = 320000
NC = 2
NS = 16
NW = NC * NS
NP = 10240
RPT = NP // NS
CH = 128
CPW = 80
EPW = CPW * CH
E_PAD = NW * EPW
PAD_DST = N + 16
RB = 2
RBS = N // RB

_mesh = plsc.VectorSubcoreMesh(
    core_axis_name="c", subcore_axis_name="s", num_cores=NC, num_subcores=NS
)


def _wid():
    return lax.axis_index("s") * NC + lax.axis_index("c")


@functools.partial(
    pl.kernel,
    out_type=jax.ShapeDtypeStruct((NC, NP), jnp.float32),
    mesh=_mesh,
    scratch_types=[
        pltpu.VMEM((CPW, CH), jnp.int32),
        pltpu.VMEM((CH,), jnp.float32),
        pltpu.VMEM((RPT,), jnp.float32),
        pltpu.VMEM_SHARED((NP,), jnp.float32),
        pltpu.SemaphoreType.DMA,
    ],
)
def _deg_kernel(dst2d, degp, idx_v, ones_v, zero_v, deg_sh, dsem):
    cid = lax.axis_index("c")
    sid = lax.axis_index("s")
    wid = _wid()
    for k in range(CH // 16):
        ones_v[pl.ds(k * 16, 16)] = jnp.ones((16,), jnp.float32)
    for k in range(RPT // 16):
        zero_v[pl.ds(k * 16, 16)] = jnp.zeros((16,), jnp.float32)
    pltpu.sync_copy(zero_v, deg_sh.at[pl.ds(sid * RPT, RPT)])
    pltpu.sync_copy(dst2d.at[pl.ds(wid * CPW, CPW)], idx_v)
    plsc.subcore_barrier()

    @pl.loop(0, CPW)
    def _chunk(j):
        pltpu.async_copy(ones_v, deg_sh.at[idx_v.at[j]], dsem, add=True)

    @pl.loop(0, CPW)
    def _drain(j):
        pltpu.make_async_copy(
            ones_v, deg_sh.at[idx_v.at[j]], dsem
        ).wait()

    plsc.subcore_barrier()
    pltpu.sync_copy(
        deg_sh.at[pl.ds(sid * RPT, RPT)], degp.at[cid, pl.ds(sid * RPT, RPT)]
    )


def _linear_body(x_ref, w_ref, degp_ref, y_ref):
    deg = degp_ref[0, :, 0] + degp_ref[1, :, 0] + 1.0
    dinv = lax.rsqrt(deg)
    xw = jnp.dot(x_ref[...], w_ref[...], preferred_element_type=jnp.float32)
    y_ref[...] = xw * dinv[:, None]


_linear = pl.pallas_call(
    _linear_body,
    grid=(RB,),
    in_specs=[
        pl.BlockSpec((RBS, C), lambda i: (i, 0)),
        pl.BlockSpec((C, C), lambda i: (0, 0)),
        pl.BlockSpec((NC, RBS, 1), lambda i: (0, i, 0)),
    ],
    out_specs=pl.BlockSpec((RBS, C), lambda i: (i, 0)),
    out_shape=jax.ShapeDtypeStruct((N, C), jnp.float32),
)


Q = 8
NG = CPW // Q
NGP = NG // 2


@functools.partial(
    pl.kernel,
    out_type=jax.ShapeDtypeStruct((NC, NP, C), jnp.float32),
    mesh=_mesh,
    scratch_types=[
        pltpu.VMEM((CPW, CH), jnp.int32),
        pltpu.VMEM((2, Q, CH), jnp.int32),
        pltpu.VMEM((2, CH, C), jnp.float32),
        pltpu.VMEM_SHARED((NP, C), jnp.float32),
        pltpu.SemaphoreType.DMA,
        pltpu.SemaphoreType.DMA,
        pltpu.SemaphoreType.DMA,
        pltpu.SemaphoreType.DMA,
        pltpu.SemaphoreType.DMA,
    ],
)
def _agg_kernel(y_hbm, src2d, dst2d, accp, sib, dib, rows, acc_sh,
                gsem0, gsem1, isem, ssem0, ssem1):
    cid = lax.axis_index("c")
    sid = lax.axis_index("s")
    wid = _wid()
    base = wid * CPW
    gsem = (gsem0, gsem1)
    ssem = (ssem0, ssem1)

    @pl.loop(0, CH)
    def _zrow(i):
        for k in range(C // 16):
            rows[0, i, pl.ds(k * 16, 16)] = jnp.zeros((16,), jnp.float32)

    for k in range(RPT // CH):
        pltpu.async_copy(
            rows.at[0], acc_sh.at[pl.ds(sid * RPT + k * CH, CH), :], ssem0
        )
    pltpu.sync_copy(src2d.at[pl.ds(base, CPW)], sib)
    pltpu.sync_copy(dst2d.at[pl.ds(base, Q)], dib.at[0])
    for k in range(RPT // CH):
        pltpu.make_async_copy(
            rows.at[0], acc_sh.at[pl.ds(sid * RPT + k * CH, CH), :], ssem0
        ).wait()
    plsc.subcore_barrier()
    pltpu.async_copy(y_hbm.at[sib.at[0]], rows.at[0], gsem0)

    @pl.loop(0, NGP)
    def _pair(gp):
        for h in range(2):
            g = gp * 2 + h
            @pl.when(g > 0)
            def _():
                pltpu.make_async_copy(
                    rows.at[1],
                    acc_sh.at[dib.at[(h + 1) % 2, Q - 1]], ssem[1]
                ).wait()
            if h == 1:
                pltpu.make_async_copy(
                    dst2d.at[pl.ds(base + g * Q, Q)], dib.at[1], isem
                ).wait()
            else:
                @pl.when(g > 0)
                def _():
                    pltpu.make_async_copy(
                        dst2d.at[pl.ds(base + g * Q, Q)], dib.at[0], isem
                    ).wait()
            @pl.when(g + 1 < NG)
            def _():
                pltpu.async_copy(
                    dst2d.at[pl.ds(base + (g + 1) * Q, Q)],
                    dib.at[(h + 1) % 2], isem,
                )
            for c in range(Q):
                t = g * Q + c
                if c > 0:
                    pltpu.make_async_copy(
                        rows.at[(c - 1) % 2],
                        acc_sh.at[dib.at[h, c - 1]], ssem[(c - 1) % 2]
                    ).wait()
                if c < Q - 1:
                    pltpu.async_copy(
                        y_hbm.at[sib.at[t + 1]],
                        rows.at[(c + 1) % 2], gsem[(c + 1) % 2],
                    )
                else:
                    @pl.when(g + 1 < NG)
                    def _():
                        pltpu.async_copy(
                            y_hbm.at[sib.at[t + 1]],
                            rows.at[(c + 1) % 2], gsem[(c + 1) % 2],
                        )
                pltpu.make_async_copy(
                    y_hbm.at[sib.at[t]], rows.at[c % 2], gsem[c % 2]
                ).wait()
                pltpu.async_copy(
                    rows.at[c % 2], acc_sh.at[dib.at[h, c]],
                    ssem[c % 2], add=True,
                )

    pltpu.make_async_copy(
        rows.at[1], acc_sh.at[dib.at[1, Q - 1]], ssem[1]
    ).wait()
    plsc.subcore_barrier()
    for k in range(RPT // CH):
        pltpu.async_copy(
            acc_sh.at[pl.ds(sid * RPT + k * CH, CH), :],
            accp.at[cid, pl.ds(sid * RPT + k * CH, CH), :],
            gsem0,
        )
    for k in range(RPT // CH):
        pltpu.make_async_copy(
            acc_sh.at[pl.ds(sid * RPT + k * CH, CH), :],
            accp.at[cid, pl.ds(sid * RPT + k * CH, CH), :],
            gsem0,
        ).wait()


def _combine_body(accp_ref, y_ref, degp_ref, b_ref, o_ref):
    deg = degp_ref[0, :, 0] + degp_ref[1, :, 0] + 1.0
    dinv = lax.rsqrt(deg)
    s = accp_ref[0] + accp_ref[1] + y_ref[...]
    o_ref[...] = s * dinv[:, None] + b_ref[...]


_combine = pl.pallas_call(
    _combine_body,
    grid=(RB,),
    in_specs=[
        pl.BlockSpec((NC, RBS, C), lambda i: (0, i, 0)),
        pl.BlockSpec((RBS, C), lambda i: (i, 0)),
        pl.BlockSpec((NC, RBS, 1), lambda i: (0, i, 0)),
        pl.BlockSpec((1, C), lambda i: (0, 0)),
    ],
    out_specs=pl.BlockSpec((RBS, C), lambda i: (i, 0)),
    out_shape=jax.ShapeDtypeStruct((N, C), jnp.float32),
)


def kernel(x, edge_index, W, b):
    src = edge_index[0].astype(jnp.int32)
    dst = edge_index[1].astype(jnp.int32)
    pad = jnp.arange(E_PAD - E, dtype=jnp.int32)
    src2d = jnp.concatenate([src, pad % N]).reshape(E_PAD // CH, CH)
    dst2d = jnp.concatenate(
        [dst, N + pad % (NP - N)]
    ).reshape(E_PAD // CH, CH)
    degp = _deg_kernel(dst2d)
    degp3 = degp.reshape(NC, NP, 1)
    y = _linear(x, W, degp3)
    accp = _agg_kernel(y, src2d, dst2d)
    return _combine(accp, y, degp3, b.reshape(1, C))


# speedup vs baseline: 1.1530x; 1.0798x over previous
import functools

import jax
import jax.numpy as jnp
from jax import lax
from jax.experimental import pallas as pl
from jax.experimental.pallas import tpu as pltpu
from jax.experimental.pallas import tpu_sc as plsc

N = 10000
C = 128
E = 320000
NC = 2
NS = 16
NW = NC * NS
NP = 10240
RPT = NP // NS
CH = 128
CPW = 80
EPW = CPW * CH
E_PAD = NW * EPW
PAD_DST = N + 16
RB = 2
RBS = N // RB

_mesh = plsc.VectorSubcoreMesh(
    core_axis_name="c", subcore_axis_name="s", num_cores=NC, num_subcores=NS
)


def _wid():
    return lax.axis_index("s") * NC + lax.axis_index("c")


@functools.partial(
    pl.kernel,
    out_type=jax.ShapeDtypeStruct((NC, NP), jnp.float32),
    mesh=_mesh,
    scratch_types=[
        pltpu.VMEM((CPW, CH), jnp.int32),
        pltpu.VMEM((CH,), jnp.float32),
        pltpu.VMEM((RPT,), jnp.float32),
        pltpu.VMEM_SHARED((NP,), jnp.float32),
        pltpu.SemaphoreType.DMA,
    ],
)
def _deg_kernel(ei2d, tail2d, degp, idx_v, ones_v, zero_v, deg_sh, dsem):
    cid = lax.axis_index("c")
    sid = lax.axis_index("s")
    wid = _wid()
    for k in range(CH // 16):
        ones_v[pl.ds(k * 16, 16)] = jnp.ones((16,), jnp.float32)
    for k in range(RPT // 16):
        zero_v[pl.ds(k * 16, 16)] = jnp.zeros((16,), jnp.float32)
    pltpu.sync_copy(zero_v, deg_sh.at[pl.ds(sid * RPT, RPT)])

    @pl.when(wid < NW - 1)
    def _():
        pltpu.sync_copy(ei2d.at[1, pl.ds(wid * CPW, CPW)], idx_v)

    @pl.when(wid == NW - 1)
    def _():
        pltpu.sync_copy(tail2d.at[1], idx_v)

    plsc.subcore_barrier()

    @pl.loop(0, CPW)
    def _chunk(j):
        pltpu.async_copy(ones_v, deg_sh.at[idx_v.at[j]], dsem, add=True)

    @pl.loop(0, CPW)
    def _drain(j):
        pltpu.make_async_copy(
            ones_v, deg_sh.at[idx_v.at[j]], dsem
        ).wait()

    plsc.subcore_barrier()
    pltpu.sync_copy(
        deg_sh.at[pl.ds(sid * RPT, RPT)], degp.at[cid, pl.ds(sid * RPT, RPT)]
    )


def _linear_body(x_ref, w_ref, degp_ref, y_ref):
    deg = degp_ref[0, :, 0] + degp_ref[1, :, 0] + 1.0
    dinv = lax.rsqrt(deg)
    xw = jnp.dot(x_ref[...], w_ref[...], preferred_element_type=jnp.float32)
    y_ref[...] = xw * dinv[:, None]


_linear = pl.pallas_call(
    _linear_body,
    grid=(RB,),
    in_specs=[
        pl.BlockSpec((RBS, C), lambda i: (i, 0)),
        pl.BlockSpec((C, C), lambda i: (0, 0)),
        pl.BlockSpec((NC, RBS, 1), lambda i: (0, i, 0)),
    ],
    out_specs=pl.BlockSpec((RBS, C), lambda i: (i, 0)),
    out_shape=jax.ShapeDtypeStruct((N, C), jnp.float32),
)


Q = 8
NG = CPW // Q
NGP = NG // 2


@functools.partial(
    pl.kernel,
    out_type=jax.ShapeDtypeStruct((NC, NP, C), jnp.float32),
    mesh=_mesh,
    scratch_types=[
        pltpu.VMEM((CPW, CH), jnp.int32),
        pltpu.VMEM((2, Q, CH), jnp.int32),
        pltpu.VMEM((2, CH, C), jnp.float32),
        pltpu.VMEM_SHARED((NP, C), jnp.float32),
        pltpu.SemaphoreType.DMA,
        pltpu.SemaphoreType.DMA,
        pltpu.SemaphoreType.DMA,
        pltpu.SemaphoreType.DMA,
        pltpu.SemaphoreType.DMA,
    ],
)
def _agg_kernel(y_hbm, ei2d, tail2d, accp, sib, dib, rows, acc_sh,
                gsem0, gsem1, isem, ssem0, ssem1):
    cid = lax.axis_index("c")
    sid = lax.axis_index("s")
    wid = _wid()
    base = wid * CPW
    gsem = (gsem0, gsem1)
    ssem = (ssem0, ssem1)
    last = NW - 1

    def _dst_group(g, buf, op):
        @pl.when(wid < last)
        def _():
            op(ei2d.at[1, pl.ds(base + g * Q, Q)], buf)

        @pl.when(wid == last)
        def _():
            op(tail2d.at[1, pl.ds(g * Q, Q)], buf)

    def _i_issue(ref, buf):
        pltpu.async_copy(ref, buf, isem)

    def _i_wait(ref, buf):
        pltpu.make_async_copy(ref, buf, isem).wait()

    @pl.loop(0, CH)
    def _zrow(i):
        for k in range(C // 16):
            rows[0, i, pl.ds(k * 16, 16)] = jnp.zeros((16,), jnp.float32)

    for k in range(RPT // CH):
        pltpu.async_copy(
            rows.at[0], acc_sh.at[pl.ds(sid * RPT + k * CH, CH), :], ssem0
        )
    @pl.when(wid < last)
    def _():
        pltpu.sync_copy(ei2d.at[0, pl.ds(base, CPW)], sib)

    @pl.when(wid == last)
    def _():
        pltpu.sync_copy(tail2d.at[0], sib)

    _dst_group(0, dib.at[0], lambda r, b: pltpu.sync_copy(r, b))
    for k in range(RPT // CH):
        pltpu.make_async_copy(
            rows.at[0], acc_sh.at[pl.ds(sid * RPT + k * CH, CH), :], ssem0
        ).wait()
    plsc.subcore_barrier()
    pltpu.async_copy(y_hbm.at[sib.at[0]], rows.at[0], gsem0)

    @pl.loop(0, NGP)
    def _pair(gp):
        for h in range(2):
            g = gp * 2 + h
            @pl.when(g > 0)
            def _():
                pltpu.make_async_copy(
                    rows.at[1],
                    acc_sh.at[dib.at[(h + 1) % 2, Q - 1]], ssem[1]
                ).wait()
            if h == 1:
                _dst_group(g, dib.at[1], _i_wait)
            else:
                @pl.when(g > 0)
                def _():
                    _dst_group(g, dib.at[0], _i_wait)
            @pl.when(g + 1 < NG)
            def _():
                _dst_group(g + 1, dib.at[(h + 1) % 2], _i_issue)
            for c in range(Q):
                t = g * Q + c
                if c > 0:
                    pltpu.make_async_copy(
                        rows.at[(c - 1) % 2],
                        acc_sh.at[dib.at[h, c - 1]], ssem[(c - 1) % 2]
                    ).wait()
                if c < Q - 1:
                    pltpu.async_copy(
                        y_hbm.at[sib.at[t + 1]],
                        rows.at[(c + 1) % 2], gsem[(c + 1) % 2],
                    )
                else:
                    @pl.when(g + 1 < NG)
                    def _():
                        pltpu.async_copy(
                            y_hbm.at[sib.at[t + 1]],
                            rows.at[(c + 1) % 2], gsem[(c + 1) % 2],
                        )
                pltpu.make_async_copy(
                    y_hbm.at[sib.at[t]], rows.at[c % 2], gsem[c % 2]
                ).wait()
                pltpu.async_copy(
                    rows.at[c % 2], acc_sh.at[dib.at[h, c]],
                    ssem[c % 2], add=True,
                )

    pltpu.make_async_copy(
        rows.at[1], acc_sh.at[dib.at[1, Q - 1]], ssem[1]
    ).wait()
    plsc.subcore_barrier()
    for k in range(RPT // CH):
        pltpu.async_copy(
            acc_sh.at[pl.ds(sid * RPT + k * CH, CH), :],
            accp.at[cid, pl.ds(sid * RPT + k * CH, CH), :],
            gsem0,
        )
    for k in range(RPT // CH):
        pltpu.make_async_copy(
            acc_sh.at[pl.ds(sid * RPT + k * CH, CH), :],
            accp.at[cid, pl.ds(sid * RPT + k * CH, CH), :],
            gsem0,
        ).wait()


def _combine_body(accp_ref, y_ref, degp_ref, b_ref, o_ref):
    deg = degp_ref[0, :, 0] + degp_ref[1, :, 0] + 1.0
    dinv = lax.rsqrt(deg)
    s = accp_ref[0] + accp_ref[1] + y_ref[...]
    o_ref[...] = s * dinv[:, None] + b_ref[...]


_combine = pl.pallas_call(
    _combine_body,
    grid=(RB,),
    in_specs=[
        pl.BlockSpec((NC, RBS, C), lambda i: (0, i, 0)),
        pl.BlockSpec((RBS, C), lambda i: (i, 0)),
        pl.BlockSpec((NC, RBS, 1), lambda i: (0, i, 0)),
        pl.BlockSpec((1, C), lambda i: (0, 0)),
    ],
    out_specs=pl.BlockSpec((RBS, C), lambda i: (i, 0)),
    out_shape=jax.ShapeDtypeStruct((N, C), jnp.float32),
)


def kernel(x, edge_index, W, b):
    ei2d = edge_index.astype(jnp.int32).reshape(2, E // CH, CH)
    pad = jnp.arange(E_PAD - E, dtype=jnp.int32)
    pads = jnp.stack(
        [pad % N, N + pad % (NP - N)]
    ).reshape(2, (E_PAD - E) // CH, CH)
    tail2d = jnp.concatenate(
        [ei2d[:, (NW - 1) * CPW:, :], pads], axis=1
    )
    degp = _deg_kernel(ei2d, tail2d)
    degp3 = degp.reshape(NC, NP, 1)
    y = _linear(x, W, degp3)
    accp = _agg_kernel(y, ei2d, tail2d)
    return _combine(accp, y, degp3, b.reshape(1, C))
